# baseline (speedup 1.0000x reference)
import functools

import jax
import jax.numpy as jnp
from jax.experimental import pallas as pl

N_HASHES = 4
Q_ATTN = 256
K_ATTN = 256


def _attn_block_kernel(q_ref, k_ref, v_ref, bo_ref, lse_ref):
    q = q_ref[0]
    k = k_ref[0]
    v = v_ref[0]
    inner = jax.lax.dot_general(q, k, (((1,), (1,)), ((), ())),
                                preferred_element_type=jnp.float32)
    m = jnp.max(inner, axis=-1, keepdims=True)
    e = jnp.exp(inner - m)
    s = jnp.sum(e, axis=-1, keepdims=True)
    bo = jax.lax.dot_general(e, v, (((1,), (0,)), ((), ())),
                             preferred_element_type=jnp.float32)
    bo_ref[0] = bo / s
    lse_ref[0, 0] = (m + jnp.log(s))[:, 0]


def _block_attention(sq, sk, sv):
    nb = sq.shape[0]
    bo, lse = pl.pallas_call(
        _attn_block_kernel,
        grid=(nb,),
        in_specs=[
            pl.BlockSpec((1, Q_ATTN, 64), lambda i: (i, 0, 0)),
            pl.BlockSpec((1, K_ATTN, 64), lambda i: (i, 0, 0)),
            pl.BlockSpec((1, K_ATTN, 64), lambda i: (i, 0, 0)),
        ],
        out_specs=[
            pl.BlockSpec((1, Q_ATTN, 64), lambda i: (i, 0, 0)),
            pl.BlockSpec((1, 1, Q_ATTN), lambda i: (i, 0, 0)),
        ],
        out_shape=[
            jax.ShapeDtypeStruct((nb, Q_ATTN, 64), jnp.float32),
            jax.ShapeDtypeStruct((nb, 1, Q_ATTN), jnp.float32),
        ],
    )(sq, sk, sv)
    return bo, lse[:, 0, :]


def _combine_kernel(o_ref, l_ref, out_ref):
    l = l_ref[:, 0, 0, :]
    m = jnp.max(l, axis=0, keepdims=True)
    w = jnp.exp(l - m)
    p = w / jnp.sum(w, axis=0, keepdims=True)
    o = o_ref[:, 0]
    out_ref[0] = jnp.sum(o * p[:, :, None], axis=0)


def _combine(o, logits):
    h, bs, n, d = o.shape
    c = 1024
    out = pl.pallas_call(
        _combine_kernel,
        grid=(bs, n // c),
        in_specs=[
            pl.BlockSpec((h, 1, c, d), lambda b, i: (0, b, i, 0)),
            pl.BlockSpec((h, 1, 1, c), lambda b, i: (0, b, 0, i)),
        ],
        out_specs=pl.BlockSpec((1, c, d), lambda b, i: (b, i, 0)),
        out_shape=jax.ShapeDtypeStruct((bs, n, d), jnp.float32),
    )(o, logits.reshape(h, bs, 1, n))
    return out


def _lsh_positions(queries, keys, alpha, beta):
    q_norms = jnp.linalg.norm(queries, axis=-1, keepdims=True)
    k_norms = jnp.linalg.norm(keys, axis=-1, keepdims=True)
    MQ = jnp.max(q_norms, axis=1, keepdims=True)
    MK = jnp.max(k_norms, axis=1, keepdims=True)
    Msq = MQ**2 + MK**2
    ext_q = jnp.sqrt(jnp.maximum(Msq - q_norms**2, 0.0))
    ext_k = jnp.sqrt(jnp.maximum(Msq - k_norms**2, 0.0))
    Q = jnp.concatenate([queries, ext_q, jnp.zeros_like(ext_q)], axis=-1)
    K = jnp.concatenate([keys, jnp.zeros_like(ext_k), ext_k], axis=-1)
    q_proj = jnp.transpose(Q @ alpha + beta, (2, 0, 1))
    k_proj = jnp.transpose(K @ alpha + beta, (2, 0, 1))
    return jnp.argsort(q_proj, axis=-1), jnp.argsort(k_proj, axis=-1)


def kernel(queries, keys, values, alpha, beta):
    bs, n, d = queries.shape
    h = N_HASHES
    q_pos, k_pos = _lsh_positions(queries, keys, alpha, beta)
    q_rev = jnp.argsort(q_pos, axis=-1)

    offs = (jnp.arange(bs, dtype=q_pos.dtype) * n)[None, :, None]
    q_flat = (q_pos + offs).reshape(-1)
    k_flat = (k_pos + offs).reshape(-1)
    sq = queries.reshape(-1, d)[q_flat].reshape(-1, Q_ATTN, d)
    sk = keys.reshape(-1, d)[k_flat].reshape(-1, K_ATTN, d)
    sv = values.reshape(-1, d)[k_flat].reshape(-1, K_ATTN, d)

    bo, lse = _block_attention(sq, sk, sv)

    offs2 = (jnp.arange(h * bs, dtype=q_rev.dtype) * n)[:, None]
    q_rev_flat = (q_rev.reshape(-1, n) + offs2).reshape(-1)
    o = bo.reshape(-1, d)[q_rev_flat].reshape(h, bs, n, d)
    logits = lse.reshape(-1)[q_rev_flat].reshape(h, bs, n)

    return _combine(o, logits)


# speedup vs baseline: 2.7117x; 2.7117x over previous
import functools

import jax
import jax.numpy as jnp
from jax import lax
from jax.experimental import pallas as pl
from jax.experimental.pallas import tpu as pltpu
from jax.experimental.pallas import tpu_sc as plsc

N_HASHES = 4
Q_ATTN = 256

_NC = 2
_NS = 16
_NW = _NC * _NS

_CHUNK = 512
_IPG = 128


def _worker_id():
    return lax.axis_index("s") * _NC + lax.axis_index("c")


def _sc_gather_body(q_hbm, k_hbm, v_hbm, qidx_hbm, kidx_hbm,
                    sq_hbm, sk_hbm, sv_hbm,
                    idx_q, idx_k, rq, rk, rv,
                    sg_q, sg_k, sg_v, sw_q, sw_k, sw_v):
    nrows = sq_hbm.shape[0]
    gpw = nrows // _NW
    nchunk = gpw // _CHUNK
    npd = _CHUNK // _IPG
    base = _worker_id() * gpw

    def chunk(c, _):
        off = base + c * _CHUNK
        pltpu.sync_copy(qidx_hbm.at[pl.ds(off, _CHUNK)], idx_q)
        pltpu.sync_copy(kidx_hbm.at[pl.ds(off, _CHUNK)], idx_k)
        gq = [pltpu.async_copy(q_hbm.at[idx_q.at[pl.ds(j * _IPG, _IPG)]],
                               rq.at[pl.ds(j * _IPG, _IPG)], sg_q)
              for j in range(npd)]
        gk = [pltpu.async_copy(k_hbm.at[idx_k.at[pl.ds(j * _IPG, _IPG)]],
                               rk.at[pl.ds(j * _IPG, _IPG)], sg_k)
              for j in range(npd)]
        gv = [pltpu.async_copy(v_hbm.at[idx_k.at[pl.ds(j * _IPG, _IPG)]],
                               rv.at[pl.ds(j * _IPG, _IPG)], sg_v)
              for j in range(npd)]
        for h in gq:
            h.wait()
        wq = pltpu.async_copy(rq, sq_hbm.at[pl.ds(off, _CHUNK)], sw_q)
        for h in gk:
            h.wait()
        wk = pltpu.async_copy(rk, sk_hbm.at[pl.ds(off, _CHUNK)], sw_k)
        for h in gv:
            h.wait()
        wv = pltpu.async_copy(rv, sv_hbm.at[pl.ds(off, _CHUNK)], sw_v)
        wq.wait()
        wk.wait()
        wv.wait()
        return 0

    lax.fori_loop(0, nchunk, chunk, 0)


def _sc_gather(queries2d, keys2d, values2d, qidx, kidx, d):
    nrows = qidx.size
    mesh = plsc.VectorSubcoreMesh(core_axis_name="c", subcore_axis_name="s")
    out = jax.ShapeDtypeStruct((nrows, d), jnp.float32)
    run = pl.kernel(
        _sc_gather_body,
        out_type=[out, out, out],
        mesh=mesh,
        scratch_types=[
            pltpu.VMEM((_CHUNK,), jnp.int32),
            pltpu.VMEM((_CHUNK,), jnp.int32),
            pltpu.VMEM((_CHUNK, d), jnp.float32),
            pltpu.VMEM((_CHUNK, d), jnp.float32),
            pltpu.VMEM((_CHUNK, d), jnp.float32),
        ] + [pltpu.SemaphoreType.DMA] * 6,
        compiler_params=pltpu.CompilerParams(use_tc_tiling_on_sc=False),
    )
    return run(queries2d, keys2d, values2d, qidx, kidx)


_SCHUNK = 1024


def _sc_scatter_body(bo_hbm, dstidx_hbm, lse_hbm, pos_hbm,
                     o_hbm, lg_hbm,
                     idx_s, rows, lbuf, pbuf, obuf,
                     sem_r, sem_s, sem_l):
    nrows = bo_hbm.shape[0]
    gpw = nrows // _NW
    nchunk = gpw // _SCHUNK
    npd = _SCHUNK // _IPG
    wid = _worker_id()
    base = wid * gpw

    def chunk(c, _):
        off = base + c * _SCHUNK
        pltpu.async_copy(bo_hbm.at[pl.ds(off, _SCHUNK)], rows, sem_r).wait()
        pltpu.sync_copy(dstidx_hbm.at[pl.ds(off // _IPG, npd)], idx_s)
        ws = [pltpu.async_copy(rows.at[pl.ds(j * _IPG, _IPG)],
                               o_hbm.at[idx_s.at[j]], sem_s)
              for j in range(npd)]
        for h in ws:
            h.wait()
        return 0

    lax.fori_loop(0, nchunk, chunk, 0)

    n = obuf.shape[0]
    rpw = gpw // n

    def lse_row(r, _):
        off = base + r * n
        pltpu.async_copy(lse_hbm.at[pl.ds(off, n)], lbuf, sem_l).wait()
        pltpu.async_copy(pos_hbm.at[pl.ds(off, n)], pbuf, sem_l).wait()

        def step(i, _):
            idx = pbuf[pl.ds(i * 16, 16)]
            val = lbuf[pl.ds(i * 16, 16)]
            plsc.store_scatter(obuf, [idx], val)
            return 0

        lax.fori_loop(0, n // 16, step, 0)
        pltpu.sync_copy(obuf, lg_hbm.at[pl.ds(off, n)])
        return 0

    lax.fori_loop(0, rpw, lse_row, 0)


def _sc_scatter(bo2d, dst_rows, lse_flat, pos_flat, n):
    nrows, d = bo2d.shape
    mesh = plsc.VectorSubcoreMesh(core_axis_name="c", subcore_axis_name="s")
    run = pl.kernel(
        _sc_scatter_body,
        out_type=[
            jax.ShapeDtypeStruct((nrows, d), jnp.float32),
            jax.ShapeDtypeStruct((nrows,), jnp.float32),
        ],
        mesh=mesh,
        scratch_types=[
            pltpu.VMEM((_SCHUNK // _IPG, _IPG), jnp.int32),
            pltpu.VMEM((_SCHUNK, d), jnp.float32),
            pltpu.VMEM((n,), jnp.float32),
            pltpu.VMEM((n,), jnp.int32),
            pltpu.VMEM((n,), jnp.float32),
        ] + [pltpu.SemaphoreType.DMA] * 3,
        compiler_params=pltpu.CompilerParams(use_tc_tiling_on_sc=False,
                                             needs_layout_passes=False),
    )
    return run(bo2d, dst_rows.reshape(-1, _IPG), lse_flat, pos_flat)


def _attn_block_kernel(q_ref, k_ref, v_ref, bo_ref, lse_ref):
    for j in range(16):
        q = q_ref[j]
        k = k_ref[j]
        v = v_ref[j]
        inner = lax.dot_general(q, k, (((1,), (1,)), ((), ())),
                                preferred_element_type=jnp.float32)
        m = jnp.max(inner, axis=-1, keepdims=True)
        e = jnp.exp(inner - m)
        s = jnp.sum(e, axis=-1, keepdims=True)
        bo = lax.dot_general(e, v, (((1,), (0,)), ((), ())),
                             preferred_element_type=jnp.float32)
        bo_ref[j] = bo / s
        lse_ref[0, 0, pl.ds(j * Q_ATTN, Q_ATTN)] = (m + jnp.log(s))[:, 0]


def _block_attention(sq, sk, sv):
    nb = sq.shape[0]
    d = sq.shape[-1]
    g = nb // 16
    bo, lse = pl.pallas_call(
        _attn_block_kernel,
        grid=(g,),
        in_specs=[
            pl.BlockSpec((16, Q_ATTN, d), lambda i: (i, 0, 0)),
            pl.BlockSpec((16, Q_ATTN, d), lambda i: (i, 0, 0)),
            pl.BlockSpec((16, Q_ATTN, d), lambda i: (i, 0, 0)),
        ],
        out_specs=[
            pl.BlockSpec((16, Q_ATTN, d), lambda i: (i, 0, 0)),
            pl.BlockSpec((1, 1, 16 * Q_ATTN), lambda i: (i, 0, 0)),
        ],
        out_shape=[
            jax.ShapeDtypeStruct((nb, Q_ATTN, d), jnp.float32),
            jax.ShapeDtypeStruct((g, 1, 16 * Q_ATTN), jnp.float32),
        ],
    )(sq, sk, sv)
    return bo, lse[:, 0, :]


def _combine_kernel(o_ref, l_ref, out_ref):
    l = l_ref[:, 0, 0, :]
    m = jnp.max(l, axis=0, keepdims=True)
    w = jnp.exp(l - m)
    p = w / jnp.sum(w, axis=0, keepdims=True)
    o = o_ref[:, 0]
    out_ref[0] = jnp.sum(o * p[:, :, None], axis=0)


def _combine(o, logits):
    h, bs, n, d = o.shape
    c = 1024
    out = pl.pallas_call(
        _combine_kernel,
        grid=(bs, n // c),
        in_specs=[
            pl.BlockSpec((h, 1, c, d), lambda b, i: (0, b, i, 0)),
            pl.BlockSpec((h, 1, 1, c), lambda b, i: (0, b, 0, i)),
        ],
        out_specs=pl.BlockSpec((1, c, d), lambda b, i: (b, i, 0)),
        out_shape=jax.ShapeDtypeStruct((bs, n, d), jnp.float32),
    )(o, logits.reshape(h, bs, 1, n))
    return out


def _lsh_positions(queries, keys, alpha, beta):
    q_norms = jnp.linalg.norm(queries, axis=-1, keepdims=True)
    k_norms = jnp.linalg.norm(keys, axis=-1, keepdims=True)
    MQ = jnp.max(q_norms, axis=1, keepdims=True)
    MK = jnp.max(k_norms, axis=1, keepdims=True)
    Msq = MQ**2 + MK**2
    ext_q = jnp.sqrt(jnp.maximum(Msq - q_norms**2, 0.0))
    ext_k = jnp.sqrt(jnp.maximum(Msq - k_norms**2, 0.0))
    Q = jnp.concatenate([queries, ext_q, jnp.zeros_like(ext_q)], axis=-1)
    K = jnp.concatenate([keys, jnp.zeros_like(ext_k), ext_k], axis=-1)
    q_proj = jnp.transpose(Q @ alpha + beta, (2, 0, 1))
    k_proj = jnp.transpose(K @ alpha + beta, (2, 0, 1))
    return jnp.argsort(q_proj, axis=-1), jnp.argsort(k_proj, axis=-1)


def kernel(queries, keys, values, alpha, beta):
    bs, n, d = queries.shape
    h = N_HASHES
    q_pos, k_pos = _lsh_positions(queries, keys, alpha, beta)

    offs = (jnp.arange(bs, dtype=jnp.int32) * n)[None, :, None]
    q_flat = (q_pos + offs).reshape(-1)
    k_flat = (k_pos + offs).reshape(-1)

    sq, sk, sv = _sc_gather(queries.reshape(-1, d), keys.reshape(-1, d),
                            values.reshape(-1, d), q_flat, k_flat, d)

    bo, lse = _block_attention(sq.reshape(-1, Q_ATTN, d),
                               sk.reshape(-1, Q_ATTN, d),
                               sv.reshape(-1, Q_ATTN, d))

    offs2 = (jnp.arange(h * bs, dtype=jnp.int32) * n)[:, None]
    dst_rows = (q_pos.reshape(-1, n) + offs2).reshape(-1)
    o_flat, logits = _sc_scatter(bo.reshape(-1, d), dst_rows,
                                 lse.reshape(-1), q_pos.reshape(-1), n)

    return _combine(o_flat.reshape(h, bs, n, d), logits.reshape(h, bs, n))


# speedup vs baseline: 3.7492x; 1.3826x over previous
import functools

import jax
import jax.numpy as jnp
from jax import lax
from jax.experimental import pallas as pl
from jax.experimental.pallas import tpu as pltpu
from jax.experimental.pallas import tpu_sc as plsc

N_HASHES = 4
Q_ATTN = 256

_NC = 2
_NS = 16
_NW = _NC * _NS

_IPG = 128
_GCHUNK = 256
_SCHUNK = 512


def _worker_id():
    return lax.axis_index("s") * _NC + lax.axis_index("c")


def _prepack_kernel(q_ref, k_ref, v_ref, qq_ref, kv_ref):
    q = q_ref[...]
    qq_ref[...] = jnp.concatenate([q, q], axis=-1)
    kv_ref[...] = jnp.concatenate([k_ref[...], v_ref[...]], axis=-1)


def _prepack(queries2d, keys2d, values2d):
    nr, d = queries2d.shape
    blk = 4096
    spec_in = pl.BlockSpec((blk, d), lambda i: (i, 0))
    spec_out = pl.BlockSpec((blk, 2 * d), lambda i: (i, 0))
    out = jax.ShapeDtypeStruct((nr, 2 * d), jnp.float32)
    return pl.pallas_call(
        _prepack_kernel,
        grid=(nr // blk,),
        in_specs=[spec_in, spec_in, spec_in],
        out_specs=[spec_out, spec_out],
        out_shape=[out, out],
    )(queries2d, keys2d, values2d)


def _sc_gather_body(qq_hbm, kv_hbm, qidx_hbm, kidx_hbm,
                    sqq_hbm, skv_hbm,
                    idx_q, idx_k, rq, rkv,
                    sg_q, sg_k, sw_q, sw_k):
    nrows = sqq_hbm.shape[0]
    gpw = nrows // _NW
    nchunk = gpw // _GCHUNK
    npd = _GCHUNK // _IPG
    base = _worker_id() * gpw

    def chunk(c, _):
        off = base + c * _GCHUNK
        pltpu.sync_copy(qidx_hbm.at[pl.ds(off, _GCHUNK)], idx_q)
        pltpu.sync_copy(kidx_hbm.at[pl.ds(off, _GCHUNK)], idx_k)
        gq = [pltpu.async_copy(qq_hbm.at[idx_q.at[pl.ds(j * _IPG, _IPG)]],
                               rq.at[pl.ds(j * _IPG, _IPG)], sg_q)
              for j in range(npd)]
        gk = [pltpu.async_copy(kv_hbm.at[idx_k.at[pl.ds(j * _IPG, _IPG)]],
                               rkv.at[pl.ds(j * _IPG, _IPG)], sg_k)
              for j in range(npd)]
        for h in gq:
            h.wait()
        wq = pltpu.async_copy(rq, sqq_hbm.at[pl.ds(off, _GCHUNK)], sw_q)
        for h in gk:
            h.wait()
        wk = pltpu.async_copy(rkv, skv_hbm.at[pl.ds(off, _GCHUNK)], sw_k)
        wq.wait()
        wk.wait()
        return 0

    lax.fori_loop(0, nchunk, chunk, 0)


def _sc_gather(qq, kv, qidx, kidx):
    nrows = qidx.size
    w = qq.shape[-1]
    mesh = plsc.VectorSubcoreMesh(core_axis_name="c", subcore_axis_name="s")
    out = jax.ShapeDtypeStruct((nrows, w), jnp.float32)
    run = pl.kernel(
        _sc_gather_body,
        out_type=[out, out],
        mesh=mesh,
        scratch_types=[
            pltpu.VMEM((_GCHUNK,), jnp.int32),
            pltpu.VMEM((_GCHUNK,), jnp.int32),
            pltpu.VMEM((_GCHUNK, w), jnp.float32),
            pltpu.VMEM((_GCHUNK, w), jnp.float32),
        ] + [pltpu.SemaphoreType.DMA] * 4,
        compiler_params=pltpu.CompilerParams(use_tc_tiling_on_sc=False),
    )
    return run(qq, kv, qidx, kidx)


def _sc_scatter_body(bo_hbm, dstidx_hbm, o_hbm, idx_s, rows, sem_r, sem_s):
    nrows = bo_hbm.shape[0]
    gpw = nrows // _NW
    nsuper = gpw // (2 * _SCHUNK)
    npd = _SCHUNK // _IPG
    base = _worker_id() * gpw

    def super_chunk(c, _):
        off0 = base + c * 2 * _SCHUNK
        pltpu.sync_copy(dstidx_hbm.at[pl.ds(off0 // _IPG, 2 * npd)], idx_s)
        for half in range(2):
            off = off0 + half * _SCHUNK
            pltpu.async_copy(bo_hbm.at[pl.ds(off, _SCHUNK)], rows,
                             sem_r).wait()
            ws = [pltpu.async_copy(rows.at[pl.ds(j * _IPG, _IPG)],
                                   o_hbm.at[idx_s.at[half * npd + j]], sem_s)
                  for j in range(npd)]
            for h in ws:
                h.wait()
        return 0

    lax.fori_loop(0, nsuper, super_chunk, 0)


def _sc_scatter(bo2d, dst_rows):
    nrows, w = bo2d.shape
    mesh = plsc.VectorSubcoreMesh(core_axis_name="c", subcore_axis_name="s")
    run = pl.kernel(
        _sc_scatter_body,
        out_type=jax.ShapeDtypeStruct((nrows, w), jnp.float32),
        mesh=mesh,
        scratch_types=[
            pltpu.VMEM((2 * _SCHUNK // _IPG, _IPG), jnp.int32),
            pltpu.VMEM((_SCHUNK, w), jnp.float32),
        ] + [pltpu.SemaphoreType.DMA] * 2,
        compiler_params=pltpu.CompilerParams(use_tc_tiling_on_sc=False),
    )
    return run(bo2d, dst_rows.reshape(-1, _IPG))


def _attn_block_kernel(qq_ref, kv_ref, bo_ref):
    for j in range(16):
        q = qq_ref[j, :, :64]
        k = kv_ref[j, :, :64]
        v = kv_ref[j, :, 64:]
        inner = lax.dot_general(q, k, (((1,), (1,)), ((), ())),
                                preferred_element_type=jnp.float32)
        m = jnp.max(inner, axis=-1, keepdims=True)
        e = jnp.exp(inner - m)
        s = jnp.sum(e, axis=-1, keepdims=True)
        bo = lax.dot_general(e, v, (((1,), (0,)), ((), ())),
                             preferred_element_type=jnp.float32)
        lse = m + jnp.log(s)
        bo_ref[j] = jnp.concatenate(
            [bo / s, jnp.broadcast_to(lse, (Q_ATTN, 64))], axis=-1)


def _block_attention(sqq, skv):
    nb = sqq.shape[0]
    g = nb // 16
    spec = pl.BlockSpec((16, Q_ATTN, 128), lambda i: (i, 0, 0))
    return pl.pallas_call(
        _attn_block_kernel,
        grid=(g,),
        in_specs=[spec, spec],
        out_specs=spec,
        out_shape=jax.ShapeDtypeStruct((nb, Q_ATTN, 128), jnp.float32),
    )(sqq, skv)


def _combine_kernel(o_ref, out_ref):
    ob = o_ref[:, 0]
    o = ob[:, :, :64]
    l = ob[:, :, 64:65]
    m = jnp.max(l, axis=0, keepdims=True)
    w = jnp.exp(l - m)
    p = w / jnp.sum(w, axis=0, keepdims=True)
    out_ref[0] = jnp.sum(o * p, axis=0)


def _combine(o_pad, bs, n, d):
    h = o_pad.shape[0]
    c = 1024
    return pl.pallas_call(
        _combine_kernel,
        grid=(bs, n // c),
        in_specs=[pl.BlockSpec((h, 1, c, 128), lambda b, i: (0, b, i, 0))],
        out_specs=pl.BlockSpec((1, c, d), lambda b, i: (b, i, 0)),
        out_shape=jax.ShapeDtypeStruct((bs, n, d), jnp.float32),
    )(o_pad)


def _lsh_positions(queries, keys, alpha, beta):
    q_norms = jnp.linalg.norm(queries, axis=-1, keepdims=True)
    k_norms = jnp.linalg.norm(keys, axis=-1, keepdims=True)
    MQ = jnp.max(q_norms, axis=1, keepdims=True)
    MK = jnp.max(k_norms, axis=1, keepdims=True)
    Msq = MQ**2 + MK**2
    ext_q = jnp.sqrt(jnp.maximum(Msq - q_norms**2, 0.0))
    ext_k = jnp.sqrt(jnp.maximum(Msq - k_norms**2, 0.0))
    Q = jnp.concatenate([queries, ext_q, jnp.zeros_like(ext_q)], axis=-1)
    K = jnp.concatenate([keys, jnp.zeros_like(ext_k), ext_k], axis=-1)
    q_proj = jnp.transpose(Q @ alpha + beta, (2, 0, 1))
    k_proj = jnp.transpose(K @ alpha + beta, (2, 0, 1))
    return jnp.argsort(q_proj, axis=-1), jnp.argsort(k_proj, axis=-1)


def kernel(queries, keys, values, alpha, beta):
    bs, n, d = queries.shape
    h = N_HASHES
    q_pos, k_pos = _lsh_positions(queries, keys, alpha, beta)

    qq, kv = _prepack(queries.reshape(-1, d), keys.reshape(-1, d),
                      values.reshape(-1, d))

    offs = (jnp.arange(bs, dtype=jnp.int32) * n)[None, :, None]
    q_flat = (q_pos + offs).reshape(-1)
    k_flat = (k_pos + offs).reshape(-1)

    sqq, skv = _sc_gather(qq, kv, q_flat, k_flat)

    bo_pad = _block_attention(sqq.reshape(-1, Q_ATTN, 2 * d),
                              skv.reshape(-1, Q_ATTN, 2 * d))

    offs2 = (jnp.arange(h * bs, dtype=jnp.int32) * n)[:, None]
    dst_rows = (q_pos.reshape(-1, n) + offs2).reshape(-1)
    o_pad = _sc_scatter(bo_pad.reshape(-1, 2 * d), dst_rows)

    return _combine(o_pad.reshape(h, bs, n, 2 * d), bs, n, d)


# speedup vs baseline: 4.6200x; 1.2323x over previous
import functools

import jax
import jax.numpy as jnp
from jax import lax
from jax.experimental import pallas as pl
from jax.experimental.pallas import tpu as pltpu
from jax.experimental.pallas import tpu_sc as plsc

N_HASHES = 4
Q_ATTN = 256

_NC = 2
_NS = 16
_NW = _NC * _NS

_IPG = 128
_GCHUNK = 256
_SCHUNK = 512


def _worker_id():
    return lax.axis_index("s") * _NC + lax.axis_index("c")


def _prepack_kernel(q_ref, k_ref, v_ref, qq_ref, kv_ref):
    q = q_ref[...]
    qq_ref[...] = jnp.concatenate([q, q], axis=-1)
    kv_ref[...] = jnp.concatenate([k_ref[...], v_ref[...]], axis=-1)


def _prepack(queries2d, keys2d, values2d):
    nr, d = queries2d.shape
    blk = 4096
    spec_in = pl.BlockSpec((blk, d), lambda i: (i, 0))
    spec_out = pl.BlockSpec((blk, 2 * d), lambda i: (i, 0))
    out = jax.ShapeDtypeStruct((nr, 2 * d), jnp.float32)
    return pl.pallas_call(
        _prepack_kernel,
        grid=(nr // blk,),
        in_specs=[spec_in, spec_in, spec_in],
        out_specs=[spec_out, spec_out],
        out_shape=[out, out],
    )(queries2d, keys2d, values2d)


def _sc_gather_body(qq_hbm, kv_hbm, qidx_hbm, kidx_hbm,
                    sqq_hbm, skv_hbm,
                    idx_q, idx_k, rq, rkv,
                    sg_q, sg_k, sw_q, sw_k):
    nrows = sqq_hbm.shape[0]
    gpw = nrows // _NW
    nchunk = gpw // _GCHUNK
    npd = _GCHUNK // _IPG
    base = _worker_id() * gpw

    def chunk(c, _):
        off = base + c * _GCHUNK
        pltpu.sync_copy(qidx_hbm.at[pl.ds(off, _GCHUNK)], idx_q)
        pltpu.sync_copy(kidx_hbm.at[pl.ds(off, _GCHUNK)], idx_k)
        gq = [pltpu.async_copy(qq_hbm.at[idx_q.at[pl.ds(j * _IPG, _IPG)]],
                               rq.at[pl.ds(j * _IPG, _IPG)], sg_q)
              for j in range(npd)]
        gk = [pltpu.async_copy(kv_hbm.at[idx_k.at[pl.ds(j * _IPG, _IPG)]],
                               rkv.at[pl.ds(j * _IPG, _IPG)], sg_k)
              for j in range(npd)]
        for h in gq:
            h.wait()
        wq = pltpu.async_copy(rq, sqq_hbm.at[pl.ds(off, _GCHUNK)], sw_q)
        for h in gk:
            h.wait()
        wk = pltpu.async_copy(rkv, skv_hbm.at[pl.ds(off, _GCHUNK)], sw_k)
        wq.wait()
        wk.wait()
        return 0

    lax.fori_loop(0, nchunk, chunk, 0)


def _sc_gather(qq, kv, qidx, kidx):
    nrows = qidx.size
    w = qq.shape[-1]
    mesh = plsc.VectorSubcoreMesh(core_axis_name="c", subcore_axis_name="s")
    out = jax.ShapeDtypeStruct((nrows, w), jnp.float32)
    run = pl.kernel(
        _sc_gather_body,
        out_type=[out, out],
        mesh=mesh,
        scratch_types=[
            pltpu.VMEM((_GCHUNK,), jnp.int32),
            pltpu.VMEM((_GCHUNK,), jnp.int32),
            pltpu.VMEM((_GCHUNK, w), jnp.float32),
            pltpu.VMEM((_GCHUNK, w), jnp.float32),
        ] + [pltpu.SemaphoreType.DMA] * 4,
        compiler_params=pltpu.CompilerParams(use_tc_tiling_on_sc=False),
    )
    return run(qq, kv, qidx, kidx)


def _sc_scatter_body(bo_hbm, dstidx_hbm, o_hbm, idx_s, rows, sem_r, sem_s):
    nrows = bo_hbm.shape[0]
    gpw = nrows // _NW
    nsuper = gpw // (2 * _SCHUNK)
    npd = _SCHUNK // _IPG
    base = _worker_id() * gpw

    def super_chunk(c, _):
        off0 = base + c * 2 * _SCHUNK
        pltpu.sync_copy(dstidx_hbm.at[pl.ds(off0 // _IPG, 2 * npd)], idx_s)
        for half in range(2):
            off = off0 + half * _SCHUNK
            pltpu.async_copy(bo_hbm.at[pl.ds(off, _SCHUNK)], rows,
                             sem_r).wait()
            ws = [pltpu.async_copy(rows.at[pl.ds(j * _IPG, _IPG)],
                                   o_hbm.at[idx_s.at[half * npd + j]], sem_s)
                  for j in range(npd)]
            for h in ws:
                h.wait()
        return 0

    lax.fori_loop(0, nsuper, super_chunk, 0)


def _sc_scatter(bo2d, dst_rows):
    nrows, w = bo2d.shape
    mesh = plsc.VectorSubcoreMesh(core_axis_name="c", subcore_axis_name="s")
    run = pl.kernel(
        _sc_scatter_body,
        out_type=jax.ShapeDtypeStruct((nrows, w), jnp.float32),
        mesh=mesh,
        scratch_types=[
            pltpu.VMEM((2 * _SCHUNK // _IPG, _IPG), jnp.int32),
            pltpu.VMEM((_SCHUNK, w), jnp.float32),
        ] + [pltpu.SemaphoreType.DMA] * 2,
        compiler_params=pltpu.CompilerParams(use_tc_tiling_on_sc=False),
    )
    return run(bo2d, dst_rows.reshape(-1, _IPG))


def _attn_block_kernel(qq_ref, kv_ref, bo_ref):
    for j in range(16):
        q = qq_ref[j, :, :64]
        k = kv_ref[j, :, :64]
        v = kv_ref[j, :, 64:]
        inner = lax.dot_general(q, k, (((1,), (1,)), ((), ())),
                                preferred_element_type=jnp.float32)
        m = jnp.max(inner, axis=-1, keepdims=True)
        e = jnp.exp(inner - m)
        s = jnp.sum(e, axis=-1, keepdims=True)
        bo = lax.dot_general(e, v, (((1,), (0,)), ((), ())),
                             preferred_element_type=jnp.float32)
        lse = m + jnp.log(s)
        bo_ref[j] = jnp.concatenate(
            [bo / s, jnp.broadcast_to(lse, (Q_ATTN, 64))], axis=-1)


def _block_attention(sqq, skv):
    nb = sqq.shape[0]
    g = nb // 16
    spec = pl.BlockSpec((16, Q_ATTN, 128), lambda i: (i, 0, 0))
    return pl.pallas_call(
        _attn_block_kernel,
        grid=(g,),
        in_specs=[spec, spec],
        out_specs=spec,
        out_shape=jax.ShapeDtypeStruct((nb, Q_ATTN, 128), jnp.float32),
    )(sqq, skv)


def _combine_kernel(o_ref, out_ref):
    ob = o_ref[:, 0]
    o = ob[:, :, :64]
    l = ob[:, :, 64:65]
    m = jnp.max(l, axis=0, keepdims=True)
    w = jnp.exp(l - m)
    p = w / jnp.sum(w, axis=0, keepdims=True)
    out_ref[0] = jnp.sum(o * p, axis=0)


def _combine(o_pad, bs, n, d):
    h = o_pad.shape[0]
    c = 1024
    return pl.pallas_call(
        _combine_kernel,
        grid=(bs, n // c),
        in_specs=[pl.BlockSpec((h, 1, c, 128), lambda b, i: (0, b, i, 0))],
        out_specs=pl.BlockSpec((1, c, d), lambda b, i: (b, i, 0)),
        out_shape=jax.ShapeDtypeStruct((bs, n, d), jnp.float32),
    )(o_pad)


def _lsh_positions(queries, keys, alpha, beta):
    q_norms = jnp.linalg.norm(queries, axis=-1, keepdims=True)
    k_norms = jnp.linalg.norm(keys, axis=-1, keepdims=True)
    MQ = jnp.max(q_norms, axis=1, keepdims=True)
    MK = jnp.max(k_norms, axis=1, keepdims=True)
    Msq = MQ**2 + MK**2
    ext_q = jnp.sqrt(jnp.maximum(Msq - q_norms**2, 0.0))
    ext_k = jnp.sqrt(jnp.maximum(Msq - k_norms**2, 0.0))
    Q = jnp.concatenate([queries, ext_q, jnp.zeros_like(ext_q)], axis=-1)
    K = jnp.concatenate([keys, jnp.zeros_like(ext_k), ext_k], axis=-1)
    q_proj = jnp.transpose(Q @ alpha + beta, (2, 0, 1))
    k_proj = jnp.transpose(K @ alpha + beta, (2, 0, 1))
    return jnp.argsort(q_proj, axis=-1), jnp.argsort(k_proj, axis=-1)


def _lsh_projections(queries, keys, alpha, beta):
    q_norms = jnp.linalg.norm(queries, axis=-1, keepdims=True)
    k_norms = jnp.linalg.norm(keys, axis=-1, keepdims=True)
    MQ = jnp.max(q_norms, axis=1, keepdims=True)
    MK = jnp.max(k_norms, axis=1, keepdims=True)
    Msq = MQ**2 + MK**2
    ext_q = jnp.sqrt(jnp.maximum(Msq - q_norms**2, 0.0))
    ext_k = jnp.sqrt(jnp.maximum(Msq - k_norms**2, 0.0))
    Q = jnp.concatenate([queries, ext_q, jnp.zeros_like(ext_q)], axis=-1)
    K = jnp.concatenate([keys, jnp.zeros_like(ext_k), ext_k], axis=-1)
    q_proj = jnp.transpose(Q @ alpha + beta, (2, 0, 1))
    k_proj = jnp.transpose(K @ alpha + beta, (2, 0, 1))
    return q_proj, k_proj


_NSPLIT = 2


def kernel(queries, keys, values, alpha, beta):
    bs, n, d = queries.shape
    h = N_HASHES
    q_proj, k_proj = _lsh_projections(queries, keys, alpha, beta)

    qq, kv = _prepack(queries.reshape(-1, d), keys.reshape(-1, d),
                      values.reshape(-1, d))

    gbs = bs // _NSPLIT
    outs = []
    for g in range(_NSPLIT):
        bsl = slice(g * gbs, (g + 1) * gbs)
        q_pos = jnp.argsort(q_proj[:, bsl], axis=-1)
        k_pos = jnp.argsort(k_proj[:, bsl], axis=-1)

        offs = ((jnp.arange(gbs, dtype=jnp.int32) + g * gbs) * n)[None, :, None]
        q_flat = (q_pos + offs).reshape(-1)
        k_flat = (k_pos + offs).reshape(-1)

        sqq, skv = _sc_gather(qq, kv, q_flat, k_flat)

        bo_pad = _block_attention(sqq.reshape(-1, Q_ATTN, 2 * d),
                                  skv.reshape(-1, Q_ATTN, 2 * d))

        offs2 = (jnp.arange(h * gbs, dtype=jnp.int32) * n)[:, None]
        dst_rows = (q_pos.reshape(-1, n) + offs2).reshape(-1)
        o_pad = _sc_scatter(bo_pad.reshape(-1, 2 * d), dst_rows)

        outs.append(_combine(o_pad.reshape(h, gbs, n, 2 * d), gbs, n, d))

    return jnp.concatenate(outs, axis=0)


# speedup vs baseline: 5.1958x; 1.1246x over previous
import functools

import jax
import jax.numpy as jnp
from jax import lax
from jax.experimental import pallas as pl
from jax.experimental.pallas import tpu as pltpu
from jax.experimental.pallas import tpu_sc as plsc

N_HASHES = 4
Q_ATTN = 256

_NC = 2
_NS = 16
_NW = _NC * _NS

_IPG = 128
_GCHUNK = 256
_SCHUNK = 512


def _worker_id():
    return lax.axis_index("s") * _NC + lax.axis_index("c")


def _prepack_kernel(q_ref, k_ref, v_ref, qq_ref, kv_ref):
    q = q_ref[...]
    qq_ref[...] = jnp.concatenate([q, q], axis=-1)
    kv_ref[...] = jnp.concatenate([k_ref[...], v_ref[...]], axis=-1)


def _prepack(queries2d, keys2d, values2d):
    nr, d = queries2d.shape
    blk = 4096
    spec_in = pl.BlockSpec((blk, d), lambda i: (i, 0))
    spec_out = pl.BlockSpec((blk, 2 * d), lambda i: (i, 0))
    out = jax.ShapeDtypeStruct((nr, 2 * d), jnp.float32)
    return pl.pallas_call(
        _prepack_kernel,
        grid=(nr // blk,),
        in_specs=[spec_in, spec_in, spec_in],
        out_specs=[spec_out, spec_out],
        out_shape=[out, out],
    )(queries2d, keys2d, values2d)


def _sc_gather_body(qq_hbm, kv_hbm, qidx_hbm, kidx_hbm,
                    sqq_hbm, skv_hbm,
                    idx_q, idx_k, rq, rkv,
                    sg_q, sg_k, sw_q, sw_k):
    nrows = sqq_hbm.shape[0]
    gpw = nrows // _NW
    nchunk = gpw // _GCHUNK
    npd = _GCHUNK // _IPG
    base = _worker_id() * gpw

    def chunk(c, _):
        off = base + c * _GCHUNK
        pltpu.sync_copy(qidx_hbm.at[pl.ds(off, _GCHUNK)], idx_q)
        pltpu.sync_copy(kidx_hbm.at[pl.ds(off, _GCHUNK)], idx_k)
        gq = [pltpu.async_copy(qq_hbm.at[idx_q.at[pl.ds(j * _IPG, _IPG)]],
                               rq.at[pl.ds(j * _IPG, _IPG)], sg_q)
              for j in range(npd)]
        gk = [pltpu.async_copy(kv_hbm.at[idx_k.at[pl.ds(j * _IPG, _IPG)]],
                               rkv.at[pl.ds(j * _IPG, _IPG)], sg_k)
              for j in range(npd)]
        for h in gq:
            h.wait()
        wq = pltpu.async_copy(rq, sqq_hbm.at[pl.ds(off, _GCHUNK)], sw_q)
        for h in gk:
            h.wait()
        wk = pltpu.async_copy(rkv, skv_hbm.at[pl.ds(off, _GCHUNK)], sw_k)
        wq.wait()
        wk.wait()
        return 0

    lax.fori_loop(0, nchunk, chunk, 0)


def _sc_gather(qq, kv, qidx, kidx):
    nrows = qidx.size
    w = qq.shape[-1]
    mesh = plsc.VectorSubcoreMesh(core_axis_name="c", subcore_axis_name="s")
    out = jax.ShapeDtypeStruct((nrows, w), jnp.float32)
    run = pl.kernel(
        _sc_gather_body,
        out_type=[out, out],
        mesh=mesh,
        scratch_types=[
            pltpu.VMEM((_GCHUNK,), jnp.int32),
            pltpu.VMEM((_GCHUNK,), jnp.int32),
            pltpu.VMEM((_GCHUNK, w), jnp.float32),
            pltpu.VMEM((_GCHUNK, w), jnp.float32),
        ] + [pltpu.SemaphoreType.DMA] * 4,
        compiler_params=pltpu.CompilerParams(use_tc_tiling_on_sc=False),
    )
    return run(qq, kv, qidx, kidx)


def _sc_scatter_body(bo_hbm, dstidx_hbm, o_hbm, idx_s, rows, sem_r, sem_s):
    nrows = bo_hbm.shape[0]
    gpw = nrows // _NW
    nsuper = gpw // (2 * _SCHUNK)
    npd = _SCHUNK // _IPG
    base = _worker_id() * gpw

    def super_chunk(c, _):
        off0 = base + c * 2 * _SCHUNK
        pltpu.sync_copy(dstidx_hbm.at[pl.ds(off0 // _IPG, 2 * npd)], idx_s)
        for half in range(2):
            off = off0 + half * _SCHUNK
            pltpu.async_copy(bo_hbm.at[pl.ds(off, _SCHUNK)], rows,
                             sem_r).wait()
            ws = [pltpu.async_copy(rows.at[pl.ds(j * _IPG, _IPG)],
                                   o_hbm.at[idx_s.at[half * npd + j]], sem_s)
                  for j in range(npd)]
            for h in ws:
                h.wait()
        return 0

    lax.fori_loop(0, nsuper, super_chunk, 0)


def _sc_scatter(bo2d, dst_rows):
    nrows, w = bo2d.shape
    mesh = plsc.VectorSubcoreMesh(core_axis_name="c", subcore_axis_name="s")
    run = pl.kernel(
        _sc_scatter_body,
        out_type=jax.ShapeDtypeStruct((nrows, w), jnp.float32),
        mesh=mesh,
        scratch_types=[
            pltpu.VMEM((2 * _SCHUNK // _IPG, _IPG), jnp.int32),
            pltpu.VMEM((_SCHUNK, w), jnp.float32),
        ] + [pltpu.SemaphoreType.DMA] * 2,
        compiler_params=pltpu.CompilerParams(use_tc_tiling_on_sc=False),
    )
    return run(bo2d, dst_rows.reshape(-1, _IPG))


def _attn_block_kernel(qq_ref, kv_ref, bo_ref):
    for j in range(16):
        q = qq_ref[j, :, :64]
        k = kv_ref[j, :, :64]
        v = kv_ref[j, :, 64:]
        inner = lax.dot_general(q, k, (((1,), (1,)), ((), ())),
                                preferred_element_type=jnp.float32)
        m = jnp.max(inner, axis=-1, keepdims=True)
        e = jnp.exp(inner - m)
        s = jnp.sum(e, axis=-1, keepdims=True)
        bo = lax.dot_general(e, v, (((1,), (0,)), ((), ())),
                             preferred_element_type=jnp.float32)
        lse = m + jnp.log(s)
        bo_ref[j] = jnp.concatenate(
            [bo / s, jnp.broadcast_to(lse, (Q_ATTN, 64))], axis=-1)


def _block_attention(sqq, skv):
    nb = sqq.shape[0]
    g = nb // 16
    spec = pl.BlockSpec((16, Q_ATTN, 128), lambda i: (i, 0, 0))
    return pl.pallas_call(
        _attn_block_kernel,
        grid=(g,),
        in_specs=[spec, spec],
        out_specs=spec,
        out_shape=jax.ShapeDtypeStruct((nb, Q_ATTN, 128), jnp.float32),
    )(sqq, skv)


def _combine_kernel(o_ref, out_ref):
    ob = o_ref[:, 0]
    o = ob[:, :, :64]
    l = ob[:, :, 64:65]
    m = jnp.max(l, axis=0, keepdims=True)
    w = jnp.exp(l - m)
    p = w / jnp.sum(w, axis=0, keepdims=True)
    out_ref[0] = jnp.sum(o * p, axis=0)


def _combine(o_pad, bs, n, d):
    h = o_pad.shape[0]
    c = 1024
    return pl.pallas_call(
        _combine_kernel,
        grid=(bs, n // c),
        in_specs=[pl.BlockSpec((h, 1, c, 128), lambda b, i: (0, b, i, 0))],
        out_specs=pl.BlockSpec((1, c, d), lambda b, i: (b, i, 0)),
        out_shape=jax.ShapeDtypeStruct((bs, n, d), jnp.float32),
    )(o_pad)


def _lsh_positions(queries, keys, alpha, beta):
    q_norms = jnp.linalg.norm(queries, axis=-1, keepdims=True)
    k_norms = jnp.linalg.norm(keys, axis=-1, keepdims=True)
    MQ = jnp.max(q_norms, axis=1, keepdims=True)
    MK = jnp.max(k_norms, axis=1, keepdims=True)
    Msq = MQ**2 + MK**2
    ext_q = jnp.sqrt(jnp.maximum(Msq - q_norms**2, 0.0))
    ext_k = jnp.sqrt(jnp.maximum(Msq - k_norms**2, 0.0))
    Q = jnp.concatenate([queries, ext_q, jnp.zeros_like(ext_q)], axis=-1)
    K = jnp.concatenate([keys, jnp.zeros_like(ext_k), ext_k], axis=-1)
    q_proj = jnp.transpose(Q @ alpha + beta, (2, 0, 1))
    k_proj = jnp.transpose(K @ alpha + beta, (2, 0, 1))
    return jnp.argsort(q_proj, axis=-1), jnp.argsort(k_proj, axis=-1)


def _lsh_projections(queries, keys, alpha, beta):
    q_norms = jnp.linalg.norm(queries, axis=-1, keepdims=True)
    k_norms = jnp.linalg.norm(keys, axis=-1, keepdims=True)
    MQ = jnp.max(q_norms, axis=1, keepdims=True)
    MK = jnp.max(k_norms, axis=1, keepdims=True)
    Msq = MQ**2 + MK**2
    ext_q = jnp.sqrt(jnp.maximum(Msq - q_norms**2, 0.0))
    ext_k = jnp.sqrt(jnp.maximum(Msq - k_norms**2, 0.0))
    Q = jnp.concatenate([queries, ext_q, jnp.zeros_like(ext_q)], axis=-1)
    K = jnp.concatenate([keys, jnp.zeros_like(ext_k), ext_k], axis=-1)
    q_proj = jnp.transpose(Q @ alpha + beta, (2, 0, 1))
    k_proj = jnp.transpose(K @ alpha + beta, (2, 0, 1))
    return q_proj, k_proj


_NSPLIT = 2


def kernel(queries, keys, values, alpha, beta):
    bs, n, d = queries.shape
    h = N_HASHES
    q_proj, k_proj = _lsh_projections(queries, keys, alpha, beta)

    qq, kv = _prepack(queries.reshape(-1, d), keys.reshape(-1, d),
                      values.reshape(-1, d))

    gbs = bs // _NSPLIT
    outs = []
    for g in range(_NSPLIT):
        bsl = slice(g * gbs, (g + 1) * gbs)
        q_pos = jnp.argsort(q_proj[:, bsl], axis=-1, stable=False)
        k_pos = jnp.argsort(k_proj[:, bsl], axis=-1, stable=False)

        offs = ((jnp.arange(gbs, dtype=jnp.int32) + g * gbs) * n)[None, :, None]
        q_flat = (q_pos + offs).reshape(-1)
        k_flat = (k_pos + offs).reshape(-1)

        sqq, skv = _sc_gather(qq, kv, q_flat, k_flat)

        bo_pad = _block_attention(sqq.reshape(-1, Q_ATTN, 2 * d),
                                  skv.reshape(-1, Q_ATTN, 2 * d))

        offs2 = (jnp.arange(h * gbs, dtype=jnp.int32) * n)[:, None]
        dst_rows = (q_pos.reshape(-1, n) + offs2).reshape(-1)
        o_pad = _sc_scatter(bo_pad.reshape(-1, 2 * d), dst_rows)

        outs.append(_combine(o_pad.reshape(h, gbs, n, 2 * d), gbs, n, d))

    return jnp.concatenate(outs, axis=0)


# speedup vs baseline: 5.2401x; 1.0085x over previous
import functools

import jax
import jax.numpy as jnp
from jax import lax
from jax.experimental import pallas as pl
from jax.experimental.pallas import tpu as pltpu
from jax.experimental.pallas import tpu_sc as plsc

N_HASHES = 4
Q_ATTN = 256

_NC = 2
_NS = 16
_NW = _NC * _NS

_IPG = 128
_GCHUNK = 256
_SCHUNK = 512


def _worker_id():
    return lax.axis_index("s") * _NC + lax.axis_index("c")


def _prepack_kernel(q_ref, k_ref, v_ref, qq_ref, kv_ref):
    q = q_ref[...]
    qq_ref[...] = jnp.concatenate([q, q], axis=-1)
    kv_ref[...] = jnp.concatenate([k_ref[...], v_ref[...]], axis=-1)


def _prepack(queries2d, keys2d, values2d):
    nr, d = queries2d.shape
    blk = 4096
    spec_in = pl.BlockSpec((blk, d), lambda i: (i, 0))
    spec_out = pl.BlockSpec((blk, 2 * d), lambda i: (i, 0))
    out = jax.ShapeDtypeStruct((nr, 2 * d), jnp.float32)
    return pl.pallas_call(
        _prepack_kernel,
        grid=(nr // blk,),
        in_specs=[spec_in, spec_in, spec_in],
        out_specs=[spec_out, spec_out],
        out_shape=[out, out],
    )(queries2d, keys2d, values2d)


def _sc_gather_body(qq_hbm, kv_hbm, qidx_hbm, kidx_hbm,
                    sqq_hbm, skv_hbm,
                    idx_q, idx_k, rq, rkv,
                    sg_q, sg_k, sw_q, sw_k):
    nrows = sqq_hbm.shape[0]
    gpw = nrows // _NW
    nchunk = gpw // _GCHUNK
    npd = _GCHUNK // _IPG
    base = _worker_id() * gpw

    def chunk(c, _):
        off = base + c * _GCHUNK
        pltpu.sync_copy(qidx_hbm.at[pl.ds(off, _GCHUNK)], idx_q)
        pltpu.sync_copy(kidx_hbm.at[pl.ds(off, _GCHUNK)], idx_k)
        gq = [pltpu.async_copy(qq_hbm.at[idx_q.at[pl.ds(j * _IPG, _IPG)]],
                               rq.at[pl.ds(j * _IPG, _IPG)], sg_q)
              for j in range(npd)]
        gk = [pltpu.async_copy(kv_hbm.at[idx_k.at[pl.ds(j * _IPG, _IPG)]],
                               rkv.at[pl.ds(j * _IPG, _IPG)], sg_k)
              for j in range(npd)]
        for h in gq:
            h.wait()
        wq = pltpu.async_copy(rq, sqq_hbm.at[pl.ds(off, _GCHUNK)], sw_q)
        for h in gk:
            h.wait()
        wk = pltpu.async_copy(rkv, skv_hbm.at[pl.ds(off, _GCHUNK)], sw_k)
        wq.wait()
        wk.wait()
        return 0

    lax.fori_loop(0, nchunk, chunk, 0)


def _sc_gather(qq, kv, qidx, kidx):
    nrows = qidx.size
    w = qq.shape[-1]
    mesh = plsc.VectorSubcoreMesh(core_axis_name="c", subcore_axis_name="s")
    out = jax.ShapeDtypeStruct((nrows, w), jnp.float32)
    run = pl.kernel(
        _sc_gather_body,
        out_type=[out, out],
        mesh=mesh,
        scratch_types=[
            pltpu.VMEM((_GCHUNK,), jnp.int32),
            pltpu.VMEM((_GCHUNK,), jnp.int32),
            pltpu.VMEM((_GCHUNK, w), jnp.float32),
            pltpu.VMEM((_GCHUNK, w), jnp.float32),
        ] + [pltpu.SemaphoreType.DMA] * 4,
        compiler_params=pltpu.CompilerParams(use_tc_tiling_on_sc=False),
    )
    return run(qq, kv, qidx, kidx)


def _sc_scatter_body(bo_hbm, dstidx_hbm, o_hbm, idx_s, rows, sem_r, sem_s):
    nrows = bo_hbm.shape[0]
    gpw = nrows // _NW
    nsuper = gpw // (2 * _SCHUNK)
    npd = _SCHUNK // _IPG
    base = _worker_id() * gpw

    def super_chunk(c, _):
        off0 = base + c * 2 * _SCHUNK
        pltpu.sync_copy(dstidx_hbm.at[pl.ds(off0 // _IPG, 2 * npd)], idx_s)
        for half in range(2):
            off = off0 + half * _SCHUNK
            pltpu.async_copy(bo_hbm.at[pl.ds(off, _SCHUNK)], rows,
                             sem_r).wait()
            ws = [pltpu.async_copy(rows.at[pl.ds(j * _IPG, _IPG)],
                                   o_hbm.at[idx_s.at[half * npd + j]], sem_s)
                  for j in range(npd)]
            for h in ws:
                h.wait()
        return 0

    lax.fori_loop(0, nsuper, super_chunk, 0)


def _sc_scatter(bo2d, dst_rows):
    nrows, w = bo2d.shape
    mesh = plsc.VectorSubcoreMesh(core_axis_name="c", subcore_axis_name="s")
    run = pl.kernel(
        _sc_scatter_body,
        out_type=jax.ShapeDtypeStruct((nrows, w), jnp.float32),
        mesh=mesh,
        scratch_types=[
            pltpu.VMEM((2 * _SCHUNK // _IPG, _IPG), jnp.int32),
            pltpu.VMEM((_SCHUNK, w), jnp.float32),
        ] + [pltpu.SemaphoreType.DMA] * 2,
        compiler_params=pltpu.CompilerParams(use_tc_tiling_on_sc=False),
    )
    return run(bo2d, dst_rows.reshape(-1, _IPG))


_ABLK = 32


def _attn_block_kernel(qq_ref, kv_ref, bo_ref):
    for j in range(_ABLK):
        q = qq_ref[j, :, :64]
        k = kv_ref[j, :, :64]
        v = kv_ref[j, :, 64:]
        inner = lax.dot_general(q, k, (((1,), (1,)), ((), ())),
                                preferred_element_type=jnp.float32)
        m = jnp.max(inner, axis=-1, keepdims=True)
        e = jnp.exp(inner - m)
        s = jnp.sum(e, axis=-1, keepdims=True)
        bo = lax.dot_general(e, v, (((1,), (0,)), ((), ())),
                             preferred_element_type=jnp.float32)
        lse = m + jnp.log(s)
        bo_ref[j] = jnp.concatenate(
            [bo / s, jnp.broadcast_to(lse, (Q_ATTN, 64))], axis=-1)


def _block_attention(sqq, skv):
    nb = sqq.shape[0]
    g = nb // _ABLK
    spec = pl.BlockSpec((_ABLK, Q_ATTN, 128), lambda i: (i, 0, 0))
    return pl.pallas_call(
        _attn_block_kernel,
        grid=(g,),
        in_specs=[spec, spec],
        out_specs=spec,
        out_shape=jax.ShapeDtypeStruct((nb, Q_ATTN, 128), jnp.float32),
    )(sqq, skv)


def _combine_kernel(o_ref, out_ref):
    ob = o_ref[:, 0]
    o = ob[:, :, :64]
    l = ob[:, :, 64:65]
    m = jnp.max(l, axis=0, keepdims=True)
    w = jnp.exp(l - m)
    p = w / jnp.sum(w, axis=0, keepdims=True)
    out_ref[0] = jnp.sum(o * p, axis=0)


def _combine(o_pad, bs, n, d):
    h = o_pad.shape[0]
    c = 4096
    return pl.pallas_call(
        _combine_kernel,
        grid=(bs, n // c),
        in_specs=[pl.BlockSpec((h, 1, c, 128), lambda b, i: (0, b, i, 0))],
        out_specs=pl.BlockSpec((1, c, d), lambda b, i: (b, i, 0)),
        out_shape=jax.ShapeDtypeStruct((bs, n, d), jnp.float32),
    )(o_pad)


def _lsh_positions(queries, keys, alpha, beta):
    q_norms = jnp.linalg.norm(queries, axis=-1, keepdims=True)
    k_norms = jnp.linalg.norm(keys, axis=-1, keepdims=True)
    MQ = jnp.max(q_norms, axis=1, keepdims=True)
    MK = jnp.max(k_norms, axis=1, keepdims=True)
    Msq = MQ**2 + MK**2
    ext_q = jnp.sqrt(jnp.maximum(Msq - q_norms**2, 0.0))
    ext_k = jnp.sqrt(jnp.maximum(Msq - k_norms**2, 0.0))
    Q = jnp.concatenate([queries, ext_q, jnp.zeros_like(ext_q)], axis=-1)
    K = jnp.concatenate([keys, jnp.zeros_like(ext_k), ext_k], axis=-1)
    q_proj = jnp.transpose(Q @ alpha + beta, (2, 0, 1))
    k_proj = jnp.transpose(K @ alpha + beta, (2, 0, 1))
    return jnp.argsort(q_proj, axis=-1), jnp.argsort(k_proj, axis=-1)


def _lsh_projections(queries, keys, alpha, beta):
    q_norms = jnp.linalg.norm(queries, axis=-1, keepdims=True)
    k_norms = jnp.linalg.norm(keys, axis=-1, keepdims=True)
    MQ = jnp.max(q_norms, axis=1, keepdims=True)
    MK = jnp.max(k_norms, axis=1, keepdims=True)
    Msq = MQ**2 + MK**2
    ext_q = jnp.sqrt(jnp.maximum(Msq - q_norms**2, 0.0))
    ext_k = jnp.sqrt(jnp.maximum(Msq - k_norms**2, 0.0))
    Q = jnp.concatenate([queries, ext_q, jnp.zeros_like(ext_q)], axis=-1)
    K = jnp.concatenate([keys, jnp.zeros_like(ext_k), ext_k], axis=-1)
    q_proj = jnp.transpose(Q @ alpha + beta, (2, 0, 1))
    k_proj = jnp.transpose(K @ alpha + beta, (2, 0, 1))
    return q_proj, k_proj


_NSPLIT = 2


def kernel(queries, keys, values, alpha, beta):
    bs, n, d = queries.shape
    h = N_HASHES
    q_proj, k_proj = _lsh_projections(queries, keys, alpha, beta)

    qq, kv = _prepack(queries.reshape(-1, d), keys.reshape(-1, d),
                      values.reshape(-1, d))

    gbs = bs // _NSPLIT
    outs = []
    for g in range(_NSPLIT):
        bsl = slice(g * gbs, (g + 1) * gbs)
        q_pos = jnp.argsort(q_proj[:, bsl], axis=-1, stable=False)
        k_pos = jnp.argsort(k_proj[:, bsl], axis=-1, stable=False)

        offs = ((jnp.arange(gbs, dtype=jnp.int32) + g * gbs) * n)[None, :, None]
        q_flat = (q_pos + offs).reshape(-1)
        k_flat = (k_pos + offs).reshape(-1)

        sqq, skv = _sc_gather(qq, kv, q_flat, k_flat)

        bo_pad = _block_attention(sqq.reshape(-1, Q_ATTN, 2 * d),
                                  skv.reshape(-1, Q_ATTN, 2 * d))

        offs2 = (jnp.arange(h * gbs, dtype=jnp.int32) * n)[:, None]
        dst_rows = (q_pos.reshape(-1, n) + offs2).reshape(-1)
        o_pad = _sc_scatter(bo_pad.reshape(-1, 2 * d), dst_rows)

        outs.append(_combine(o_pad.reshape(h, gbs, n, 2 * d), gbs, n, d))

    return jnp.concatenate(outs, axis=0)
